# baseline (speedup 1.0000x reference)
import functools

import jax
import jax.numpy as jnp
from jax import lax
from jax.experimental import pallas as pl
from jax.experimental.pallas import tpu as pltpu
from jax.experimental.pallas import tpu_sc as plsc

NUM_SPACE = 8
CONCEPT_DIM = 32
NUM_CONCEPT = 1024
TOK_TILE = 224
NUM_TOK_TILES = 7
NUM_TOKENS = TOK_TILE * NUM_TOK_TILES
NUM_ROWS = NUM_TOKENS * NUM_SPACE


def _mm_kt(a, b):
    return jax.lax.dot_general(
        a, b, (((1,), (1,)), ((), ())), preferred_element_type=jnp.float32)


def _split2(v):
    hi = v.astype(jnp.bfloat16)
    lo = (v - hi.astype(jnp.float32)).astype(jnp.bfloat16)
    return hi, lo


def _argmin_rows(d):
    val = d
    idx = jax.lax.broadcasted_iota(jnp.int32, d.shape, 0)
    r = d.shape[0]
    while r > 1:
        h = r // 2
        a, b = val[:h], val[h:]
        ia, ib = idx[:h], idx[h:]
        m = b < a
        val = jnp.where(m, b, a)
        idx = jnp.where(m, ib, ia)
        r = h
    return idx[0]


def _argmin_body(x_ref, e_ref, zk_ref, ea_ref, en_ref):
    @pl.when(pl.program_id(0) == 0)
    def _prep():
        for i in range(NUM_SPACE):
            E = e_ref[i]
            eh, el = _split2(E)
            et = (E - eh.astype(jnp.float32)
                  - el.astype(jnp.float32)).astype(jnp.bfloat16)
            ea_ref[i] = jnp.concatenate([eh, eh, el, el, eh, et], axis=1)
            en_ref[i] = jnp.sum(E * E, axis=1, keepdims=True)

    xf = x_ref[0]
    xh, xl = _split2(xf)
    xt = (xf - xh.astype(jnp.float32)
          - xl.astype(jnp.float32)).astype(jnp.bfloat16)
    zks = []
    for i in range(NUM_SPACE):
        sl = slice(i * CONCEPT_DIM, (i + 1) * CONCEPT_DIM)
        zc = jnp.concatenate([xh[:, sl], xl[:, sl]], axis=1)
        za = jnp.concatenate([zc, zc, xt[:, sl], xh[:, sl]], axis=1)
        scores = _mm_kt(ea_ref[i], za)
        d = en_ref[i] - 2.0 * scores
        zks.append(jnp.argmin(d, axis=0).astype(jnp.int32))
    zk_ref[0] = jnp.stack(zks, axis=0)


GATHER_CHUNKS = 4
CHUNK = 98
PAD_D = 128


def _sc_gather(table_hbm, idx_hbm, out_hbm, idx_v, rows_v, sem):
    info = plsc.get_sparse_core_info()
    wid = lax.axis_index("s") * info.num_cores + lax.axis_index("c")
    pltpu.sync_copy(idx_hbm.at[wid], idx_v)
    copies = [
        pltpu.async_copy(table_hbm.at[idx_v.at[c]], rows_v.at[c], sem)
        for c in range(GATHER_CHUNKS)
    ]
    for cp in copies:
        cp.wait()
    pltpu.sync_copy(rows_v, out_hbm.at[wid])


def kernel(x, embeds):
    B, H, W, C = x.shape
    x3 = x.reshape(NUM_TOK_TILES, TOK_TILE, C)
    zk = pl.pallas_call(
        _argmin_body,
        grid=(NUM_TOK_TILES,),
        in_specs=[
            pl.BlockSpec((1, TOK_TILE, C), lambda j: (j, 0, 0)),
            pl.BlockSpec((NUM_SPACE, NUM_CONCEPT, CONCEPT_DIM),
                         lambda j: (0, 0, 0)),
        ],
        out_specs=pl.BlockSpec((1, NUM_SPACE, TOK_TILE), lambda j: (j, 0, 0)),
        out_shape=jax.ShapeDtypeStruct(
            (NUM_TOK_TILES, NUM_SPACE, TOK_TILE), jnp.int32),
        scratch_shapes=[
            pltpu.VMEM((NUM_SPACE, NUM_CONCEPT, 6 * CONCEPT_DIM),
                       jnp.bfloat16),
            pltpu.VMEM((NUM_SPACE, NUM_CONCEPT, 1), jnp.float32),
        ],
    )(x3, embeds)
    idx = (zk + (jnp.arange(NUM_SPACE, dtype=jnp.int32)[None, :, None]
                 * NUM_CONCEPT)).transpose(0, 2, 1)

    info = plsc.get_sparse_core_info()
    nw = info.num_cores * info.num_subcores
    mesh = plsc.VectorSubcoreMesh(core_axis_name="c", subcore_axis_name="s")
    gather = functools.partial(
        pl.kernel, mesh=mesh,
        out_type=jax.ShapeDtypeStruct((nw, GATHER_CHUNKS, CHUNK, PAD_D),
                                      jnp.float32),
        scratch_types=[
            pltpu.VMEM((GATHER_CHUNKS, CHUNK), jnp.int32),
            pltpu.VMEM((GATHER_CHUNKS, CHUNK, PAD_D), jnp.float32),
            pltpu.SemaphoreType.DMA,
        ],
    )(_sc_gather)
    table_pad = jnp.pad(
        embeds.reshape(NUM_SPACE * NUM_CONCEPT, CONCEPT_DIM),
        ((0, 0), (0, PAD_D - CONCEPT_DIM)))
    rows = gather(table_pad, idx.reshape(nw, GATHER_CHUNKS, CHUNK))

    z_q = rows.reshape(NUM_ROWS, PAD_D)[:, :CONCEPT_DIM].reshape(B, H, W, C)
    z_k = zk.transpose(1, 0, 2).reshape(NUM_SPACE, B, H * W)
    return (z_q, x, z_k)


# speedup vs baseline: 1.0785x; 1.0785x over previous
import functools

import jax
import jax.numpy as jnp
from jax import lax
from jax.experimental import pallas as pl
from jax.experimental.pallas import tpu as pltpu
from jax.experimental.pallas import tpu_sc as plsc

NUM_SPACE = 8
CONCEPT_DIM = 32
NUM_CONCEPT = 1024
TOK_TILE = 224
NUM_TOK_TILES = 7
NUM_TOKENS = TOK_TILE * NUM_TOK_TILES
NUM_ROWS = NUM_TOKENS * NUM_SPACE


def _mm_kt(a, b):
    return jax.lax.dot_general(
        a, b, (((1,), (1,)), ((), ())), preferred_element_type=jnp.float32)


def _split2(v):
    hi = v.astype(jnp.bfloat16)
    lo = (v - hi.astype(jnp.float32)).astype(jnp.bfloat16)
    return hi, lo


def _argmin_rows(d):
    val = d
    idx = jax.lax.broadcasted_iota(jnp.int32, d.shape, 0)
    r = d.shape[0]
    while r > 1:
        h = r // 2
        a, b = val[:h], val[h:]
        ia, ib = idx[:h], idx[h:]
        m = b < a
        val = jnp.where(m, b, a)
        idx = jnp.where(m, ib, ia)
        r = h
    return idx[0]


def _argmin_body(x_ref, e_ref, zk_ref, ea_ref):
    @pl.when(pl.program_id(0) == 0)
    def _prep():
        for i in range(NUM_SPACE):
            E = e_ref[i]
            eh, el = _split2(E)
            et = (E - eh.astype(jnp.float32)
                  - el.astype(jnp.float32)).astype(jnp.bfloat16)
            en = jnp.sum(E * E, axis=1, keepdims=True)
            enh, enl = _split2(en)
            ent = (en - enh.astype(jnp.float32)
                   - enl.astype(jnp.float32)).astype(jnp.bfloat16)
            ea_ref[i] = jnp.concatenate(
                [eh, eh, el, el, eh, et, enh, enl, ent], axis=1)

    xf = x_ref[0]
    xh, xl = _split2(xf)
    xt = (xf - xh.astype(jnp.float32)
          - xl.astype(jnp.float32)).astype(jnp.bfloat16)
    half = jnp.full((TOK_TILE, 3), -0.5, jnp.bfloat16)
    zks = []
    for i in range(NUM_SPACE):
        sl = slice(i * CONCEPT_DIM, (i + 1) * CONCEPT_DIM)
        zc = jnp.concatenate([xh[:, sl], xl[:, sl]], axis=1)
        za = jnp.concatenate([zc, zc, xt[:, sl], xh[:, sl], half], axis=1)
        scores = _mm_kt(ea_ref[i], za)
        zks.append(jnp.argmax(scores, axis=0).astype(jnp.int32))
    zk_ref[0] = jnp.stack(zks, axis=0)


GATHER_CHUNKS = 4
CHUNK = 98
PAD_D = 128


def _sc_gather(table_hbm, idx_hbm, out_hbm, idx_v, rows_v, sem):
    info = plsc.get_sparse_core_info()
    wid = lax.axis_index("s") * info.num_cores + lax.axis_index("c")
    pltpu.sync_copy(idx_hbm.at[wid], idx_v)
    copies = [
        pltpu.async_copy(table_hbm.at[idx_v.at[c]], rows_v.at[c], sem)
        for c in range(GATHER_CHUNKS)
    ]
    for cp in copies:
        cp.wait()
    pltpu.sync_copy(rows_v, out_hbm.at[wid])


def kernel(x, embeds):
    B, H, W, C = x.shape
    x3 = x.reshape(NUM_TOK_TILES, TOK_TILE, C)
    zk = pl.pallas_call(
        _argmin_body,
        grid=(NUM_TOK_TILES,),
        in_specs=[
            pl.BlockSpec((1, TOK_TILE, C), lambda j: (j, 0, 0)),
            pl.BlockSpec((NUM_SPACE, NUM_CONCEPT, CONCEPT_DIM),
                         lambda j: (0, 0, 0)),
        ],
        out_specs=pl.BlockSpec((1, NUM_SPACE, TOK_TILE), lambda j: (j, 0, 0)),
        out_shape=jax.ShapeDtypeStruct(
            (NUM_TOK_TILES, NUM_SPACE, TOK_TILE), jnp.int32),
        scratch_shapes=[
            pltpu.VMEM((NUM_SPACE, NUM_CONCEPT, 6 * CONCEPT_DIM + 3),
                       jnp.bfloat16),
        ],
    )(x3, embeds)
    idx = (zk + (jnp.arange(NUM_SPACE, dtype=jnp.int32)[None, :, None]
                 * NUM_CONCEPT)).transpose(0, 2, 1)

    info = plsc.get_sparse_core_info()
    nw = info.num_cores * info.num_subcores
    mesh = plsc.VectorSubcoreMesh(core_axis_name="c", subcore_axis_name="s")
    gather = functools.partial(
        pl.kernel, mesh=mesh,
        out_type=jax.ShapeDtypeStruct((nw, GATHER_CHUNKS, CHUNK, PAD_D),
                                      jnp.float32),
        scratch_types=[
            pltpu.VMEM((GATHER_CHUNKS, CHUNK), jnp.int32),
            pltpu.VMEM((GATHER_CHUNKS, CHUNK, PAD_D), jnp.float32),
            pltpu.SemaphoreType.DMA,
        ],
    )(_sc_gather)
    table_pad = jnp.pad(
        embeds.reshape(NUM_SPACE * NUM_CONCEPT, CONCEPT_DIM),
        ((0, 0), (0, PAD_D - CONCEPT_DIM)))
    rows = gather(table_pad, idx.reshape(nw, GATHER_CHUNKS, CHUNK))

    z_q = rows.reshape(NUM_ROWS, PAD_D)[:, :CONCEPT_DIM].reshape(B, H, W, C)
    z_k = zk.transpose(1, 0, 2).reshape(NUM_SPACE, B, H * W)
    return (z_q, x, z_k)
